# Initial kernel scaffold; baseline (speedup 1.0000x reference)
import jax
import jax.numpy as jnp
from jax.experimental import pallas as pl


def kernel(node_embedding, nbr_idx):
    raise NotImplementedError("write your pallas kernel here")


# speedup vs baseline: 1035.6474x; 1035.6474x over previous
import functools

import jax
import jax.numpy as jnp
from jax import lax
from jax.experimental import pallas as pl
from jax.experimental.pallas import tpu as pltpu
from jax.experimental.pallas import tpu_sc as plsc

B, At, Nbr, D = 2, 512, 24, 64
NC, NS = 2, 16
AP = At // NS
NBT = AP * Nbr
CB = 32
NCH = NBT // CB


def _mesh():
    return plsc.VectorSubcoreMesh(
        core_axis_name="c", subcore_axis_name="s", num_cores=NC, num_subcores=NS
    )


@functools.partial(
    pl.kernel,
    out_type=jax.ShapeDtypeStruct((B, At * Nbr, D), jnp.float32),
    mesh=_mesh(),
    scratch_types=[
        pltpu.VMEM((NBT,), jnp.int32),
        pltpu.VMEM((NBT, D), jnp.float32),
        pltpu.SemaphoreType.DMA,
    ],
    compiler_params=pltpu.CompilerParams(use_tc_tiling_on_sc=False),
)
def _gather_neighbors(emb, nbr, gat, tref, buf, sem):
    c = lax.axis_index("c")
    s = lax.axis_index("s")
    base = s * NBT
    pltpu.sync_copy(nbr.at[c, pl.ds(base, NBT)], tref)
    pltpu.async_copy(emb.at[c].at[tref], buf, sem).wait()
    pltpu.sync_copy(buf, gat.at[c, pl.ds(base, NBT)])


@functools.partial(
    pl.kernel,
    out_type=jax.ShapeDtypeStruct((B, At * Nbr, Nbr * D), jnp.float32),
    mesh=_mesh(),
    scratch_types=[
        pltpu.VMEM((CB,), jnp.int32),
        pltpu.VMEM((CB, Nbr * D), jnp.float32),
        pltpu.SemaphoreType.DMA,
    ],
)
def _gather_blocks(gat, nbr, out, idxv, buf, sem):
    c = lax.axis_index("c")
    s = lax.axis_index("s")
    for i in range(NCH):
        q0 = s * NBT + i * CB
        pltpu.sync_copy(nbr.at[c, pl.ds(q0, CB)], idxv)
        pltpu.async_copy(gat.at[c].at[idxv], buf, sem).wait()
        pltpu.sync_copy(buf, out.at[c, pl.ds(q0, CB)])


def kernel(node_embedding, nbr_idx):
    b, at, d = node_embedding.shape
    nbr = nbr_idx.shape[2]
    assert (b, at, nbr, d) == (B, At, Nbr, D)
    nbr_flat = nbr_idx.astype(jnp.int32).reshape(b, at * nbr)
    gat = _gather_neighbors(node_embedding, nbr_flat)
    gat_blocks = gat.reshape(b, at, nbr * d)
    out = _gather_blocks(gat_blocks, nbr_flat)
    return out.reshape(b, at, nbr, nbr, d)


# speedup vs baseline: 1091.3860x; 1.0538x over previous
import functools

import jax
import jax.numpy as jnp
from jax import lax
from jax.experimental import pallas as pl
from jax.experimental.pallas import tpu as pltpu
from jax.experimental.pallas import tpu_sc as plsc

B, At, Nbr, D = 2, 512, 24, 64
NC, NS = 2, 16
AP = At // NS
NBT = AP * Nbr
CB = 32
NCH = NBT // CB


def _mesh():
    return plsc.VectorSubcoreMesh(
        core_axis_name="c", subcore_axis_name="s", num_cores=NC, num_subcores=NS
    )


@functools.partial(
    pl.kernel,
    out_type=jax.ShapeDtypeStruct((B, At * Nbr, D), jnp.float32),
    mesh=_mesh(),
    scratch_types=[
        pltpu.VMEM((NBT,), jnp.int32),
        pltpu.VMEM((NBT, D), jnp.float32),
        pltpu.SemaphoreType.DMA,
    ],
    compiler_params=pltpu.CompilerParams(use_tc_tiling_on_sc=False),
)
def _gather_neighbors(emb, nbr, gat, tref, buf, sem):
    c = lax.axis_index("c")
    s = lax.axis_index("s")
    base = s * NBT
    pltpu.sync_copy(nbr.at[c, pl.ds(base, NBT)], tref)
    pltpu.async_copy(emb.at[c].at[tref], buf, sem).wait()
    pltpu.sync_copy(buf, gat.at[c, pl.ds(base, NBT)])


@functools.partial(
    pl.kernel,
    out_type=jax.ShapeDtypeStruct((B, At, Nbr, Nbr, D), jnp.float32),
    mesh=_mesh(),
    scratch_types=[
        pltpu.VMEM((NBT,), jnp.int32),
        pltpu.VMEM((Nbr, Nbr, D), jnp.float32),
        pltpu.VMEM((Nbr, Nbr, D), jnp.float32),
        pltpu.SemaphoreType.DMA,
        pltpu.SemaphoreType.DMA,
    ],
    compiler_params=pltpu.CompilerParams(use_tc_tiling_on_sc=False),
)
def _gather_blocks(gat, nbr, out, tref, buf0, buf1, sem0, sem1):
    c = lax.axis_index("c")
    s = lax.axis_index("s")
    a0 = s * AP
    bufs, sems = (buf0, buf1), (sem0, sem1)

    pltpu.sync_copy(nbr.at[c, pl.ds(a0 * Nbr, NBT)], tref)
    cur = pltpu.async_copy(gat.at[c].at[tref.at[pl.ds(0, Nbr)]], bufs[0], sems[0])
    for i in range(AP):
        p, q = i & 1, (i + 1) & 1
        if i + 1 < AP:
            nxt = pltpu.async_copy(
                gat.at[c].at[tref.at[pl.ds((i + 1) * Nbr, Nbr)]], bufs[q], sems[q]
            )
        cur.wait()
        pltpu.sync_copy(bufs[p], out.at[c, a0 + i])
        if i + 1 < AP:
            cur = nxt


def kernel(node_embedding, nbr_idx):
    b, at, d = node_embedding.shape
    nbr = nbr_idx.shape[2]
    assert (b, at, nbr, d) == (B, At, Nbr, D)
    nbr_flat = nbr_idx.astype(jnp.int32).reshape(b, at * nbr)
    gat = _gather_neighbors(node_embedding, nbr_flat)
    gat_atoms = gat.reshape(b, at, nbr, d)
    return _gather_blocks(gat_atoms, nbr_flat)
